# Initial kernel scaffold; baseline (speedup 1.0000x reference)
import jax
import jax.numpy as jnp
from jax.experimental import pallas as pl


def kernel(nodes, edge_index, edge_attr, edge_type_mask, g1_Wm1, g1_bm1, g1_Wm2, g1_bm2, g1_Wu, g1_bu, g2_Wm1, g2_bm1, g2_Wm2, g2_bm2, g2_Wu, g2_bu, a_W1, a_b1, a_W2, a_b2):
    raise NotImplementedError("write your pallas kernel here")


# speedup vs baseline: 2.5157x; 2.5157x over previous
import functools

import jax
import jax.numpy as jnp
from jax import lax
from jax.experimental import pallas as pl
from jax.experimental.pallas import tpu as pltpu
from jax.experimental.pallas import tpu_sc as plsc

N = 10000
NP = 10240
E = 320000
ED = 16
ESEL = 160000

NC = 2
NS = 16
NW = NC * NS

F32 = jnp.float32
BF16 = jnp.bfloat16


_EB = 2000


def _edge_mlp_kernel(r_ref, s_ref, e_ref, w256_ref, we_ref, b1_ref,
                     w2_ref, b2_ref, o0_ref, o1_ref):
    cat = jnp.concatenate([r_ref[...], s_ref[...]], axis=-1).astype(BF16)
    pre = ((jnp.dot(cat, w256_ref[...], preferred_element_type=F32)
            + jnp.dot(e_ref[...].astype(BF16), we_ref[...],
                      preferred_element_type=F32))
           + b1_ref[...])
    m = jnp.maximum(pre, 0.0).astype(BF16)
    msg = jnp.dot(m, w2_ref[...], preferred_element_type=F32) + b2_ref[...]
    o0_ref[...] = msg[:, :64]
    o1_ref[...] = msg[:, 64:]


def _edge_mlp(r, s, e, w1, b1, w2, b2):
    return pl.pallas_call(
        _edge_mlp_kernel,
        grid=(E // _EB,),
        in_specs=[
            pl.BlockSpec((_EB, 128), lambda i: (i, 0)),
            pl.BlockSpec((_EB, 128), lambda i: (i, 0)),
            pl.BlockSpec((_EB, ED), lambda i: (i, 0)),
            pl.BlockSpec((256, 128), lambda i: (0, 0)),
            pl.BlockSpec((ED, 128), lambda i: (0, 0)),
            pl.BlockSpec((1, 128), lambda i: (0, 0)),
            pl.BlockSpec((128, 128), lambda i: (0, 0)),
            pl.BlockSpec((1, 128), lambda i: (0, 0)),
        ],
        out_specs=[
            pl.BlockSpec((_EB, 64), lambda i: (i, 0)),
            pl.BlockSpec((_EB, 64), lambda i: (i, 0)),
        ],
        out_shape=[
            jax.ShapeDtypeStruct((E, 64), F32),
            jax.ShapeDtypeStruct((E, 64), F32),
        ],
    )(r, s, e, w1[:256], w1[256:], b1.reshape(1, 128), w2,
      b2.reshape(1, 128))


def _nodeupd_kernel(x_ref, s0_ref, s1_ref, wu_ref, bu_ref, h_ref):
    cat = jnp.concatenate([x_ref[...], s0_ref[...], s1_ref[...]],
                          axis=-1).astype(BF16)
    h_ref[...] = jnp.maximum(
        jnp.dot(cat, wu_ref[...], preferred_element_type=F32) + bu_ref[...],
        0.0)


def _nodeupd(x, s_tab, wu, bu, nb=1280):
    nblk = NP // nb
    return pl.pallas_call(
        _nodeupd_kernel,
        grid=(nblk,),
        in_specs=[
            pl.BlockSpec((nb, 128), lambda i: (i, 0)),
            pl.BlockSpec((nb, 64), lambda i: (i, 0)),
            pl.BlockSpec((nb, 64), lambda i, _n=nblk: (_n + i, 0)),
            pl.BlockSpec((256, 128), lambda i: (0, 0)),
            pl.BlockSpec((1, 128), lambda i: (0, 0)),
        ],
        out_specs=pl.BlockSpec((nb, 128), lambda i: (i, 0)),
        out_shape=jax.ShapeDtypeStruct((NP, 128), F32),
    )(x, s_tab, s_tab, wu, bu.reshape(1, 128))


def _actor_mlp_kernel(r_ref, s_ref, e_ref, w256_ref, we_ref, b1_ref,
                      w2_ref, b2_ref, pm_ref, o_ref):
    cat = jnp.concatenate([r_ref[...], s_ref[...]], axis=-1).astype(BF16)
    pre = ((jnp.dot(cat, w256_ref[...], preferred_element_type=F32)
            + jnp.dot(e_ref[...].astype(BF16), we_ref[...],
                      preferred_element_type=F32))
           + b1_ref[...])
    m = jnp.maximum(pre, 0.0).astype(BF16)
    o2 = jnp.dot(m, w2_ref[...], preferred_element_type=F32) + b2_ref[...]
    op = 0.5 * lax.dot_general(
        pm_ref[...], o2, dimension_numbers=(((1,), (0,)), ((), ())),
        preferred_element_type=F32, precision=lax.Precision.HIGHEST)
    mean = op[:, 0:1]
    std = jnp.exp(jnp.clip(op[:, 1:2], -20.0, 2.0))
    o_ref[...] = jnp.concatenate([mean, std, op[:, 2:]], axis=-1)


def _actor_mlp(r, s, e, w1, b1, w2, b2):
    w2p = jnp.zeros((128, 8), F32).at[:, 0:2].set(w2)
    b2p = jnp.zeros((1, 8), F32).at[0, 0:2].set(b2)
    pm = jnp.zeros((_EB // 2, _EB), F32)
    pm = pm.at[jnp.arange(_EB // 2), 2 * jnp.arange(_EB // 2)].set(1.0)
    pm = pm.at[jnp.arange(_EB // 2), 2 * jnp.arange(_EB // 2) + 1].set(1.0)
    return pl.pallas_call(
        _actor_mlp_kernel,
        grid=(ESEL // _EB,),
        in_specs=[
            pl.BlockSpec((_EB, 128), lambda i: (i, 0)),
            pl.BlockSpec((_EB, 128), lambda i: (i, 0)),
            pl.BlockSpec((_EB, ED), lambda i: (i, 0)),
            pl.BlockSpec((256, 128), lambda i: (0, 0)),
            pl.BlockSpec((ED, 128), lambda i: (0, 0)),
            pl.BlockSpec((1, 128), lambda i: (0, 0)),
            pl.BlockSpec((128, 8), lambda i: (0, 0)),
            pl.BlockSpec((1, 8), lambda i: (0, 0)),
            pl.BlockSpec((_EB // 2, _EB), lambda i: (0, 0)),
        ],
        out_specs=pl.BlockSpec((_EB // 2, 8), lambda i: (i, 0)),
        out_shape=jax.ShapeDtypeStruct((ESEL // 2, 8), F32),
    )(r, s, e, w1[:256], w1[256:], b1.reshape(1, 128), w2p, b2p, pm)


_MESH = plsc.VectorSubcoreMesh(
    core_axis_name="c", subcore_axis_name="s", num_cores=NC, num_subcores=NS)

_IW = 80
_NSUB = 5
_EK = _IW * _NSUB
_EPW = E // NW
_EPT = E // NS
_ROWS_PT = NP // NS
_RREM = _ROWS_PT - _EK
_UNTILED = pltpu.CompilerParams(use_tc_tiling_on_sc=False)

_GATHER_SCRATCH = [
    pltpu.VMEM((_NSUB, _IW), jnp.int32),
    pltpu.VMEM((_NSUB, _IW), jnp.int32),
    pltpu.VMEM((_EK, 128), F32),
    pltpu.VMEM((_EK, 128), F32),
    pltpu.SemaphoreType.DMA,
    pltpu.SemaphoreType.DMA,
]


def _gather2_kernel(x_hbm, ir_hbm, is_hbm, r_out, s_out,
                    idxr_v, idxs_v, r_v, s_v, sem_r, sem_s):
    c = lax.axis_index("c")
    tid = lax.axis_index("s")
    wid = tid * NC + c

    def chunk(i, _):
        base = wid * _EPW + i * _EK
        rb = base // _IW
        pltpu.sync_copy(ir_hbm.at[pl.ds(rb, _NSUB)], idxr_v)
        pltpu.sync_copy(is_hbm.at[pl.ds(rb, _NSUB)], idxs_v)
        cps = []
        for j in range(_NSUB):
            dst = pl.ds(j * _IW, _IW)
            cps.append(pltpu.async_copy(
                x_hbm.at[idxr_v.at[j]], r_v.at[dst], sem_r))
            cps.append(pltpu.async_copy(
                x_hbm.at[idxs_v.at[j]], s_v.at[dst], sem_s))
        for cp in cps:
            cp.wait()
        pltpu.sync_copy(r_v, r_out.at[pl.ds(base, _EK)])
        pltpu.sync_copy(s_v, s_out.at[pl.ds(base, _EK)])
        return 0
    lax.fori_loop(0, _EPW // _EK, chunk, 0)


_gather2 = functools.partial(
    pl.kernel,
    out_type=[
        jax.ShapeDtypeStruct((E, 128), F32),
        jax.ShapeDtypeStruct((E, 128), F32),
    ],
    mesh=_MESH,
    scratch_types=_GATHER_SCRATCH,
    compiler_params=_UNTILED,
)(_gather2_kernel)


_SCAT_SCRATCH = [
    pltpu.VMEM((_NSUB, _IW), jnp.int32),
    pltpu.VMEM((_EK, 64), F32),
    pltpu.VMEM_SHARED((NP, 64), F32),
    pltpu.SemaphoreType.DMA,
]


def _scatter64_kernel(m0_hbm, m1_hbm, ir_hbm, zn_hbm,
                      s_out,
                      idxr_v, m_v, s_sh, sem_m):
    c = lax.axis_index("c")
    tid = lax.axis_index("s")

    row0 = tid * _ROWS_PT
    pltpu.sync_copy(zn_hbm, m_v)
    pltpu.sync_copy(m_v, s_sh.at[pl.ds(row0, _EK)])
    pltpu.sync_copy(m_v.at[pl.ds(0, _RREM)],
                    s_sh.at[pl.ds(row0 + _EK, _RREM)])
    plsc.subcore_barrier()

    def chunk(i, _):
        base = tid * _EPT + i * _EK
        rb = base // _IW
        pltpu.sync_copy(ir_hbm.at[pl.ds(rb, _NSUB)], idxr_v)

        @pl.when(c == 0)
        def _():
            pltpu.sync_copy(m0_hbm.at[pl.ds(base, _EK)], m_v)

        @pl.when(c == 1)
        def _():
            pltpu.sync_copy(m1_hbm.at[pl.ds(base, _EK)], m_v)

        for j in range(_NSUB):
            pltpu.sync_copy(m_v.at[pl.ds(j * _IW, _IW)],
                            s_sh.at[idxr_v.at[j]], add=True)
        return 0
    lax.fori_loop(0, _EPT // _EK, chunk, 0)
    plsc.subcore_barrier()

    pltpu.sync_copy(s_sh.at[pl.ds(row0, _EK)], m_v)
    pltpu.sync_copy(m_v, s_out.at[pl.ds(c * NP + row0, _EK)])
    pltpu.sync_copy(s_sh.at[pl.ds(row0 + _EK, _RREM)],
                    m_v.at[pl.ds(0, _RREM)])
    pltpu.sync_copy(m_v.at[pl.ds(0, _RREM)],
                    s_out.at[pl.ds(c * NP + row0 + _EK, _RREM)])


_scatter64 = functools.partial(
    pl.kernel,
    out_type=jax.ShapeDtypeStruct((2 * NP, 64), F32),
    mesh=_MESH,
    scratch_types=_SCAT_SCRATCH,
    compiler_params=_UNTILED,
)(_scatter64_kernel)


_NCHUNKS = ESEL // _EK

_ACTOR_SCRATCH = [
    pltpu.VMEM((_NSUB, _IW), jnp.int32),
    pltpu.VMEM((_NSUB, _IW), jnp.int32),
    pltpu.VMEM((_NSUB, _IW), jnp.int32),
    pltpu.VMEM((_EK, 128), F32),
    pltpu.VMEM((_EK, 128), F32),
    pltpu.VMEM((_EK, ED), F32),
    pltpu.SemaphoreType.DMA,
    pltpu.SemaphoreType.DMA,
    pltpu.SemaphoreType.DMA,
    pltpu.SemaphoreType.DMA,
    pltpu.SemaphoreType.DMA,
]


def _actor_gather_kernel(h_hbm, ea_hbm, ir_hbm, is_hbm, mask_hbm,
                         r_out, s_out, e_out,
                         mask_v, ir_v, is_v, r_v, s_v, e_v,
                         sem_a, sem_b, sem_c, sem_d, sem_e):
    c = lax.axis_index("c")
    s = lax.axis_index("s")
    wid = s * NC + c
    nch = (_NCHUNKS - 1 - wid) // NW + 1

    def chunk(jj, _):
        ci = wid + jj * NW
        pltpu.sync_copy(mask_hbm.at[pl.ds(ci * _NSUB, _NSUB)], mask_v)
        gps = [pltpu.async_copy(ir_hbm.at[mask_v.at[j]], ir_v.at[j],
                                sem_d if (j % 2 == 0) else sem_e)
               for j in range(_NSUB)]
        for gp in gps:
            gp.wait()
        gps = [pltpu.async_copy(is_hbm.at[mask_v.at[j]], is_v.at[j],
                                sem_d if (j % 2 == 0) else sem_e)
               for j in range(_NSUB)]
        for gp in gps:
            gp.wait()

        cps = []
        for j in range(_NSUB):
            dst = pl.ds(j * _IW, _IW)
            cps.append(pltpu.async_copy(
                h_hbm.at[ir_v.at[j]], r_v.at[dst], sem_a))
            cps.append(pltpu.async_copy(
                h_hbm.at[is_v.at[j]], s_v.at[dst], sem_b))
            cps.append(pltpu.async_copy(
                ea_hbm.at[mask_v.at[j]], e_v.at[dst], sem_c))
        for cp in cps:
            cp.wait()

        base = ci * _EK
        pltpu.sync_copy(r_v, r_out.at[pl.ds(base, _EK)])
        pltpu.sync_copy(s_v, s_out.at[pl.ds(base, _EK)])
        pltpu.sync_copy(e_v, e_out.at[pl.ds(base, _EK)])
        return 0
    lax.fori_loop(0, nch, chunk, 0)


_actor_gather = functools.partial(
    pl.kernel,
    out_type=[
        jax.ShapeDtypeStruct((ESEL, 128), F32),
        jax.ShapeDtypeStruct((ESEL, 128), F32),
        jax.ShapeDtypeStruct((ESEL, ED), F32),
    ],
    mesh=_MESH,
    scratch_types=_ACTOR_SCRATCH,
    compiler_params=_UNTILED,
)(_actor_gather_kernel)


def kernel(nodes, edge_index, edge_attr, edge_type_mask,
           g1_Wm1, g1_bm1, g1_Wm2, g1_bm2, g1_Wu, g1_bu,
           g2_Wm1, g2_bm1, g2_Wm2, g2_bm2, g2_Wu, g2_bu,
           a_W1, a_b1, a_W2, a_b2):
    x = jnp.pad(nodes[0], ((0, NP - N), (0, 0)))
    ea = edge_attr[0]
    idx_r = edge_index[0].astype(jnp.int32)
    idx_s = edge_index[1].astype(jnp.int32)
    mask = edge_type_mask.astype(jnp.int32)
    zn = jnp.zeros((_EK, 64), F32)

    ir2 = idx_r.reshape(E // _IW, _IW)
    is2 = idx_s.reshape(E // _IW, _IW)
    mask2 = mask.reshape(ESEL // _IW, _IW)

    def layer(xl, Wm1, bm1, Wm2, bm2, Wu, bu):
        r, s = _gather2(xl, ir2, is2)
        m0, m1 = _edge_mlp(r, s, ea, Wm1, bm1, Wm2, bm2)
        s_tab = _scatter64(m0, m1, ir2, zn)
        return _nodeupd(xl, s_tab, Wu, bu)

    h = layer(x, g1_Wm1, g1_bm1, g1_Wm2, g1_bm2, g1_Wu, g1_bu)
    h = layer(h, g2_Wm1, g2_bm1, g2_Wm2, g2_bm2, g2_Wu, g2_bu)

    hr, hs, easel = _actor_gather(h, ea, idx_r, idx_s, mask2)
    o = _actor_mlp(hr, hs, easel, a_W1, a_b1, a_W2, a_b2)
    return (o[:, 0].reshape(1, -1), o[:, 1].reshape(1, -1))
